# baseline (speedup 1.0000x reference)
import functools

import jax
import jax.numpy as jnp
from jax import lax
from jax.experimental import pallas as pl
from jax.experimental.pallas import tpu as pltpu
from jax.experimental.pallas import tpu_sc as plsc

NE = 8
LANES = 16
NWORK = 32


def _prologue_body(w_ref, b_ref, x_ref, p_ref, bsum_ref, wsum_ref):
    g = pl.program_id(0)
    f = w_ref.shape[1]
    i_in = f // NE

    @pl.when(g == 0)
    def _():
        row = lax.broadcasted_iota(jnp.int32, (f, i_in), 0)
        col = lax.broadcasted_iota(jnp.int32, (f, i_in), 1)
        fold = (row % i_in == col).astype(jnp.float32)
        parts = []
        for e in range(NE):
            cs = jnp.sum(w_ref[e * i_in:(e + 1) * i_in, :], axis=0,
                         keepdims=True)
            parts.append(lax.dot_general(
                cs, fold, (((1,), (0,)), ((), ())),
                precision=lax.Precision.HIGHEST,
                preferred_element_type=jnp.float32))
        wsum_ref[...] = jnp.concatenate(parts, axis=1)
        bsum_ref[...] = jnp.sum(b_ref[...], keepdims=True).reshape(1, 1)

    xw = x_ref[...] * wsum_ref[...]
    seg_r = lax.broadcasted_iota(jnp.int32, (f, NE), 0)
    seg_c = lax.broadcasted_iota(jnp.int32, (f, NE), 1)
    seg = (seg_r // i_in == seg_c).astype(jnp.float32)
    p_ref[...] = lax.dot_general(xw, seg, (((1,), (0,)), ((), ())),
                                 precision=lax.Precision.HIGHEST,
                                 preferred_element_type=jnp.float32)


def _pass_a_body(p_ref, d_ref, bsum_ref, s_ref, *, nt):
    g = pl.program_id(0)
    cap = s_ref.shape[2]

    @pl.when(g % nt == 0)
    def _():
        s_ref[0] = jnp.broadcast_to(bsum_ref[...], (NE, cap))

    p = p_ref[...]
    incs = [
        lax.dot_general(p[:, e:e + 1], d_ref[:, e, :],
                        (((0,), (0,)), ((), ())),
                        precision=lax.Precision.DEFAULT,
                        preferred_element_type=jnp.float32)
        for e in range(NE)
    ]
    s_ref[0] += jnp.concatenate(incs, axis=0)


def _pass_b_body(c_ref, s_ref, o_ref):
    acc = lax.dot_general(c_ref[:, 0, :], s_ref[0, 0:1, :],
                          (((1,), (1,)), ((), ())),
                          precision=lax.Precision.DEFAULT,
                          preferred_element_type=jnp.float32)
    for e in range(1, NE):
        acc += lax.dot_general(c_ref[:, e, :], s_ref[0, e:e + 1, :],
                               (((1,), (1,)), ((), ())),
                               precision=lax.Precision.DEFAULT,
                               preferred_element_type=jnp.float32)
    o_ref[...] = acc


def _lane_reduce_body(pp_ref, o_ref):
    o_ref[...] = jnp.sum(pp_ref[...], axis=1, keepdims=True)


def _make_sc_pass_b(t_tok, rt, cap, nr):
    nchunk = nr // LANES
    mesh = plsc.VectorSubcoreMesh(core_axis_name="c", subcore_axis_name="s")

    @functools.partial(
        pl.kernel,
        mesh=mesh,
        out_type=jax.ShapeDtypeStruct((2 * (t_tok - rt) * LANES,),
                                      jnp.float32),
        scratch_types=[
            pltpu.VMEM((LANES, NE, cap), jnp.float32),
            pltpu.VMEM((NE, cap), jnp.float32),
            pltpu.VMEM((nr * LANES,), jnp.float32),
        ],
    )
    def sc_kernel(c_hbm, s_hbm, outp_hbm, cbuf, sbuf, obuf):
        wid = lax.axis_index("s") * 2 + lax.axis_index("c")
        b_w = wid // 16
        sub = wid % 16
        row0 = b_w * t_tok + rt + sub * nr
        out0 = (b_w * (t_tok - rt) + sub * nr) * LANES
        pltpu.sync_copy(s_hbm.at[b_w], sbuf)
        njv = (NE * cap) // LANES // NE
        for ch in range(nchunk):
            pltpu.sync_copy(c_hbm.at[pl.ds(row0 + ch * LANES, LANES)], cbuf)

            def jbody(j, accs):
                se = [sbuf[e, pl.ds(j * LANES, LANES)] for e in range(NE)]
                new = []
                for r in range(LANES):
                    a = accs[r]
                    for e in range(NE):
                        a = a + cbuf[r, e, pl.ds(j * LANES, LANES)] * se[e]
                    new.append(a)
                return tuple(new)

            accs = lax.fori_loop(
                0, njv, jbody,
                tuple(jnp.zeros((LANES,), jnp.float32)
                      for _ in range(LANES)))
            for r in range(LANES):
                obuf[pl.ds((ch * LANES + r) * LANES, LANES)] = accs[r]
        pltpu.sync_copy(obuf, outp_hbm.at[pl.ds(out0, nr * LANES)])

    return sc_kernel


def kernel(x, combine_array, dispatch_mask, weight, bias):
    b, t, f = x.shape
    e, c = dispatch_mask.shape[2], dispatch_mask.shape[3]
    assert e == NE and b == 2
    i_in = f // e
    tblk = 512
    bt = b * t
    nt = t // tblk
    nbt = bt // tblk
    rt = 1536
    nt_tc = rt // tblk
    nr = (t - rt) // 16

    x2 = x.reshape(bt, f)
    d3 = dispatch_mask.reshape(bt, e, c)
    c3 = combine_array.reshape(bt, e, c)

    p, bsum, _ = pl.pallas_call(
        _prologue_body,
        grid=(nbt,),
        in_specs=[
            pl.BlockSpec((f, f), lambda g: (0, 0)),
            pl.BlockSpec((1, f), lambda g: (0, 0)),
            pl.BlockSpec((tblk, f), lambda g: (g, 0)),
        ],
        out_specs=[
            pl.BlockSpec((tblk, NE), lambda g: (g, 0)),
            pl.BlockSpec((1, 1), lambda g: (0, 0)),
            pl.BlockSpec((1, f), lambda g: (0, 0)),
        ],
        out_shape=[
            jax.ShapeDtypeStruct((bt, NE), jnp.float32),
            jax.ShapeDtypeStruct((1, 1), jnp.float32),
            jax.ShapeDtypeStruct((1, f), jnp.float32),
        ],
    )(weight, bias.reshape(1, f), x2)

    s = pl.pallas_call(
        functools.partial(_pass_a_body, nt=nt),
        grid=(nbt,),
        in_specs=[
            pl.BlockSpec((tblk, NE), lambda g: (g, 0)),
            pl.BlockSpec((tblk, e, c), lambda g: (g, 0, 0)),
            pl.BlockSpec((1, 1), lambda g: (0, 0)),
        ],
        out_specs=pl.BlockSpec((1, e, c), lambda g, nt=nt: (g // nt, 0, 0)),
        out_shape=jax.ShapeDtypeStruct((b, e, c), jnp.float32),
    )(p, d3, bsum)

    out_tc = pl.pallas_call(
        _pass_b_body,
        grid=(b, nt_tc),
        in_specs=[
            pl.BlockSpec((tblk, e, c),
                         lambda gb, gt, nt=nt: (gb * nt + gt, 0, 0)),
            pl.BlockSpec((1, e, c), lambda gb, gt: (gb, 0, 0)),
        ],
        out_specs=pl.BlockSpec(
            (tblk, 1), lambda gb, gt, n=nt_tc: (gb * n + gt, 0)),
        out_shape=jax.ShapeDtypeStruct((b * rt, 1), jnp.float32),
    )(c3, s)

    outp = _make_sc_pass_b(t, rt, c, nr)(c3, s)

    out_sc = pl.pallas_call(
        _lane_reduce_body,
        grid=(1,),
        in_specs=[pl.BlockSpec((b * (t - rt), 16), lambda g: (0, 0))],
        out_specs=pl.BlockSpec((b * (t - rt), 1), lambda g: (0, 0)),
        out_shape=jax.ShapeDtypeStruct((b * (t - rt), 1), jnp.float32),
    )(outp.reshape(b * (t - rt), 16))

    out = jnp.concatenate(
        [out_tc.reshape(b, rt), out_sc.reshape(b, t - rt)], axis=1)
    return out
